# baseline (speedup 1.0000x reference)
import functools

import jax
import jax.numpy as jnp
from jax import lax
from jax.experimental import pallas as pl
from jax.experimental.pallas import tpu as pltpu
from jax.experimental.pallas import tpu_sc as plsc

N = 10000
NPAD = 10240
D = 1024
H2 = 512
F = 128
C = D // F
NC, NS, L = 2, 16, 16
NW = NC * NS
E_TRAIN = 160000
E_PAD = 163840
E_SCORE = 100000
EB = 16
NB2 = 208
E2_PAD = 32 * EB * NB2

ROW_BLK = 640

_mesh = plsc.VectorSubcoreMesh(core_axis_name="c", subcore_axis_name="s")


NBH = E_PAD // (NC * NS * 128)
NROWS_T = NPAD // NS


@functools.partial(
    pl.kernel,
    out_type=jax.ShapeDtypeStruct((NC * NPAD, F), jnp.float32),
    mesh=_mesh,
    scratch_types=[
        pltpu.VMEM((NBH, 128), jnp.int32),
        pltpu.VMEM((128, F), jnp.float32),
        pltpu.VMEM_SHARED((NPAD, F), jnp.float32),
    ],
)
def _hist_k(dst_hbm, ones_hbm, zeros_hbm, out_hbm, didx_v, ones_v, deg_sh):
    cid = lax.axis_index("c")
    tid = lax.axis_index("s")
    pltpu.sync_copy(dst_hbm.at[pl.ds((cid * NS + tid) * NBH, NBH)], didx_v)
    pltpu.sync_copy(ones_hbm, ones_v)
    pltpu.sync_copy(zeros_hbm, deg_sh.at[pl.ds(tid * NROWS_T, NROWS_T)])
    plsc.subcore_barrier()

    def body(b, _):
        pltpu.sync_copy(ones_v, deg_sh.at[didx_v.at[b]], add=True)
        return 0

    lax.fori_loop(0, NBH, body, 0)
    plsc.subcore_barrier()
    pltpu.sync_copy(deg_sh.at[pl.ds(tid * NROWS_T, NROWS_T)],
                    out_hbm.at[pl.ds(cid * NPAD + tid * NROWS_T, NROWS_T)])


def _prep_body(parts_ref, x_ref, xs_ref, dinv_ref):
    p = parts_ref[:, :, 0:1]
    deg = p[0] + p[1] + 1.0
    dinv = lax.rsqrt(deg)
    xsb = x_ref[...] * dinv
    for c in range(C):
        xs_ref[c] = xsb[:, c * F:(c + 1) * F]
    dinv_ref[...] = dinv


def _prep(parts, x):
    return pl.pallas_call(
        _prep_body,
        grid=(NPAD // ROW_BLK,),
        in_specs=[
            pl.BlockSpec((NC, ROW_BLK, F), lambda i: (0, i, 0)),
            pl.BlockSpec((ROW_BLK, D), lambda i: (i, 0)),
        ],
        out_specs=[
            pl.BlockSpec((C, ROW_BLK, F), lambda i: (0, i, 0)),
            pl.BlockSpec((ROW_BLK, 1), lambda i: (i, 0)),
        ],
        out_shape=[
            jax.ShapeDtypeStruct((C, NPAD, F), jnp.float32),
            jax.ShapeDtypeStruct((NPAD, 1), jnp.float32),
        ],
    )(parts, x)


EPS = E_PAD // NS
EBS = 64
NB = EPS // EBS
NB8 = 16
NBUF = 3


@functools.partial(
    pl.kernel,
    out_type=jax.ShapeDtypeStruct((C * NPAD, F), jnp.float32),
    mesh=_mesh,
    scratch_types=[
        pltpu.VMEM((NB8, EBS), jnp.int32),
        pltpu.VMEM((NB8, EBS), jnp.int32),
        pltpu.VMEM((NBUF, EBS, F), jnp.float32),
        pltpu.VMEM_SHARED((NPAD, F), jnp.float32),
        pltpu.SemaphoreType.DMA,
        pltpu.SemaphoreType.DMA,
        pltpu.SemaphoreType.DMA,
    ],
)
def _scatter_k(xs_hbm, srcadj_hbm, dst_hbm, zeros_hbm, out_hbm,
               src_v, dst_v, rows_v, z_sh, g0, g1, g2):
    cid = lax.axis_index("c")
    tid = lax.axis_index("s")
    gs = (g0, g1, g2)
    CPS = C // NC

    for cc in range(CPS):
        c = cid * CPS + cc
        pltpu.sync_copy(zeros_hbm, z_sh.at[pl.ds(tid * NROWS_T, NROWS_T)])
        plsc.subcore_barrier()

        def block(h, _):
            pltpu.sync_copy(
                srcadj_hbm.at[pl.ds(c * (E_PAD // EBS) + tid * NB + h * NB8,
                                    NB8)],
                src_v)
            pltpu.sync_copy(dst_hbm.at[pl.ds(tid * NB + h * NB8, NB8)], dst_v)

            def wait_gather(b, buf):
                pltpu.make_async_copy(xs_hbm.at[src_v.at[b]],
                                      rows_v.at[buf], gs[buf]).wait()

            for b in range(NBUF):
                pltpu.async_copy(xs_hbm.at[src_v.at[b]],
                                 rows_v.at[b], gs[b])
            for b in range(NB8):
                buf = b % NBUF
                wait_gather(b, buf)
                pltpu.sync_copy(rows_v.at[buf], z_sh.at[dst_v.at[b]],
                                add=True)
                if b + NBUF < NB8:
                    pltpu.async_copy(xs_hbm.at[src_v.at[b + NBUF]],
                                     rows_v.at[buf], gs[buf])
            return 0

        lax.fori_loop(0, NB // NB8, block, 0)
        plsc.subcore_barrier()
        pltpu.sync_copy(
            z_sh.at[pl.ds(tid * NROWS_T, NROWS_T)],
            out_hbm.at[pl.ds(c * NPAD + tid * NROWS_T, NROWS_T)])


def _head_body(z_ref, xs_ref, dinv_ref, x_ref,
               wcat_ref, bcat_ref, wlt_ref, wlb_ref, bl_ref, out_ref):
    y = jnp.concatenate(
        [z_ref[c] + xs_ref[c] for c in range(C)], axis=-1) * dinv_ref[...]
    g = jnp.maximum(jnp.dot(y, wcat_ref[...], preferred_element_type=jnp.float32)
                    + bcat_ref[...], 0.0)
    h = (jnp.dot(g, wlt_ref[...], preferred_element_type=jnp.float32)
         + jnp.dot(x_ref[...], wlb_ref[...], preferred_element_type=jnp.float32)
         + bl_ref[...])
    out_ref[...] = jnp.maximum(h, 0.0).astype(jnp.bfloat16)


def _head(z, xs, dinv, x, wcat, bcat, wl_top, wl_bot, bl):
    return pl.pallas_call(
        _head_body,
        grid=(NPAD // ROW_BLK,),
        in_specs=[
            pl.BlockSpec((C, ROW_BLK, F), lambda i: (0, i, 0)),
            pl.BlockSpec((C, ROW_BLK, F), lambda i: (0, i, 0)),
            pl.BlockSpec((ROW_BLK, 1), lambda i: (i, 0)),
            pl.BlockSpec((ROW_BLK, D), lambda i: (i, 0)),
            pl.BlockSpec((D, D), lambda i: (0, 0)),
            pl.BlockSpec((1, D), lambda i: (0, 0)),
            pl.BlockSpec((D, H2), lambda i: (0, 0)),
            pl.BlockSpec((D, H2), lambda i: (0, 0)),
            pl.BlockSpec((1, H2), lambda i: (0, 0)),
        ],
        out_specs=pl.BlockSpec((ROW_BLK, H2), lambda i: (i, 0)),
        out_shape=jax.ShapeDtypeStruct((NPAD, H2), jnp.bfloat16),
    )(z, xs, dinv, x, wcat, bcat.reshape(1, D), wl_top, wl_bot,
      bl.reshape(1, H2))


EB_T = E2_PAD // NW
H2I = H2 // 2
NBUF2 = 4
LOOK = 2


@functools.partial(
    pl.kernel,
    out_type=[
        jax.ShapeDtypeStruct((E2_PAD, H2I), jnp.int32),
        jax.ShapeDtypeStruct((E2_PAD, H2I), jnp.int32),
    ],
    mesh=_mesh,
    scratch_types=[
        pltpu.VMEM((NB2, EB), jnp.int32),
        pltpu.VMEM((NB2, EB), jnp.int32),
        pltpu.VMEM((NBUF2, EB, H2I), jnp.int32),
        pltpu.VMEM((NBUF2, EB, H2I), jnp.int32),
    ] + [pltpu.SemaphoreType.DMA] * 16,
)
def _score_k(h2_hbm, sidx_hbm, didx_hbm, hs_hbm, hd_hbm,
             sidx_v, didx_v, srows_v, drows_v, *sems):
    wid = lax.axis_index("s") * NC + lax.axis_index("c")
    pltpu.sync_copy(sidx_hbm.at[pl.ds(wid * NB2, NB2)], sidx_v)
    pltpu.sync_copy(didx_hbm.at[pl.ds(wid * NB2, NB2)], didx_v)
    gssem = sems[0:4]
    gdsem = sems[4:8]
    wssem = sems[8:12]
    wdsem = sems[12:16]
    base = wid * EB_T

    def g_src(b, s):
        return pltpu.make_async_copy(h2_hbm.at[sidx_v.at[b]],
                                     srows_v.at[s], gssem[s])

    def g_dst(b, s):
        return pltpu.make_async_copy(h2_hbm.at[didx_v.at[b]],
                                     drows_v.at[s], gdsem[s])

    def w_src(b, s):
        return pltpu.make_async_copy(srows_v.at[s],
                                     hs_hbm.at[pl.ds(base + b * EB, EB)],
                                     wssem[s])

    def w_dst(b, s):
        return pltpu.make_async_copy(drows_v.at[s],
                                     hd_hbm.at[pl.ds(base + b * EB, EB)],
                                     wdsem[s])

    for b in range(LOOK):
        g_src(b, b).start()
        g_dst(b, b).start()

    def body(j, _):
        for k in range(NBUF2):
            b = NBUF2 * j + k
            g_src(b, k).wait()
            w_src(b, k).start()
            g_dst(b, k).wait()
            w_dst(b, k).start()
            sq = (k + LOOK) % NBUF2

            @pl.when(b >= LOOK)
            def _():
                w_src(b - LOOK, sq).wait()
                w_dst(b - LOOK, sq).wait()

            @pl.when(b + LOOK < NB2)
            def _():
                g_src(b + LOOK, sq).start()
                g_dst(b + LOOK, sq).start()
        return 0

    lax.fori_loop(0, NB2 // NBUF2, body, 0)
    for b in range(NB2 - LOOK, NB2):
        w_src(b, b % NBUF2).wait()
        w_dst(b, b % NBUF2).wait()


EROW = 2048


def _edot_body(hs_ref, hd_ref, out_ref):
    ones = jnp.ones((H2I, 1), jnp.float32)
    acc = jnp.zeros((EROW, 1), jnp.float32)
    s = hs_ref[...]
    d = hd_ref[...]
    for part in (jnp.int32(-65536), None):
        if part is None:
            sf = lax.bitcast_convert_type(lax.shift_left(s, 16), jnp.float32)
            df = lax.bitcast_convert_type(lax.shift_left(d, 16), jnp.float32)
        else:
            sf = lax.bitcast_convert_type(s & part, jnp.float32)
            df = lax.bitcast_convert_type(d & part, jnp.float32)
        acc = acc + jnp.dot(sf * df, ones, preferred_element_type=jnp.float32)
    out_ref[...] = acc


def _edot(hs, hd):
    return pl.pallas_call(
        _edot_body,
        grid=(E2_PAD // EROW,),
        in_specs=[
            pl.BlockSpec((EROW, H2I), lambda i: (i, 0)),
            pl.BlockSpec((EROW, H2I), lambda i: (i, 0)),
        ],
        out_specs=pl.BlockSpec((EROW, 1), lambda i: (i, 0)),
        out_shape=jax.ShapeDtypeStruct((E2_PAD, 1), jnp.float32),
    )(hs, hd)


def kernel(pos_edge_index, neg_edge_index, x, train_pos_edge_index,
           W1, b1, W2, b2, W3, b3, W4, b4, Wl, bl):
    i32 = jnp.int32
    src = train_pos_edge_index[0].astype(i32)
    dst = train_pos_edge_index[1].astype(i32)
    padN = jnp.full((E_PAD - E_TRAIN,), N, i32)
    src_p = jnp.concatenate([src, padN])
    dst_p = jnp.concatenate([dst, padN])
    srcadj = (src_p[None, :] + (jnp.arange(C, dtype=i32) * NPAD)[:, None])
    srcadj = srcadj.reshape(C * (E_PAD // EBS), EBS)
    dst2d = dst_p.reshape(E_PAD // 128, 128)
    dst2d_s = dst_p.reshape(E_PAD // EBS, EBS)
    zeros_rows = jnp.zeros((NROWS_T, F), jnp.float32)

    ones_rows = jnp.ones((128, F), jnp.float32)
    parts = _hist_k(dst2d, ones_rows, zeros_rows).reshape(NC, NPAD, F)
    x_pad = jnp.pad(x, ((0, NPAD - N), (0, 0)))
    xs, dinv = _prep(parts, x_pad)

    z_flat = _scatter_k(xs.reshape(C * NPAD, F), srcadj, dst2d_s, zeros_rows)
    z = z_flat.reshape(C, NPAD, F)

    wcat = jnp.concatenate([W1, W2, W3, W4], axis=1)
    bcat = jnp.concatenate([b1, b2, b3, b4])
    h2 = _head(z, xs, dinv, x_pad, wcat, bcat, Wl[:D], Wl[D:], bl)

    h2i = lax.bitcast_convert_type(h2.reshape(NPAD, H2 // 2, 2), i32)

    te = jnp.concatenate([pos_edge_index, neg_edge_index], axis=-1).astype(i32)
    pad0 = jnp.zeros((E2_PAD - E_SCORE,), i32)
    te_src = jnp.concatenate([te[0], pad0]).reshape(E2_PAD // EB, EB)
    te_dst = jnp.concatenate([te[1], pad0]).reshape(E2_PAD // EB, EB)
    hs, hd = _score_k(h2i, te_src, te_dst)
    scores = _edot(hs, hd)
    return scores[:E_SCORE, 0]


# speedup vs baseline: 1.0198x; 1.0198x over previous
import functools

import jax
import jax.numpy as jnp
from jax import lax
from jax.experimental import pallas as pl
from jax.experimental.pallas import tpu as pltpu
from jax.experimental.pallas import tpu_sc as plsc

N = 10000
NPAD = 10240
D = 1024
H2 = 512
F = 128
C = D // F
NC, NS, L = 2, 16, 16
NW = NC * NS
E_TRAIN = 160000
E_PAD = 163840
E_SCORE = 100000
EB = 32
NB2 = 104
E2_PAD = 32 * EB * NB2

ROW_BLK = 640

_mesh = plsc.VectorSubcoreMesh(core_axis_name="c", subcore_axis_name="s")


NBH = E_PAD // (NC * NS * 128)
NROWS_T = NPAD // NS


@functools.partial(
    pl.kernel,
    out_type=jax.ShapeDtypeStruct((NC * NPAD, F), jnp.float32),
    mesh=_mesh,
    scratch_types=[
        pltpu.VMEM((NBH, 128), jnp.int32),
        pltpu.VMEM((128, F), jnp.float32),
        pltpu.VMEM_SHARED((NPAD, F), jnp.float32),
    ],
)
def _hist_k(dst_hbm, ones_hbm, zeros_hbm, out_hbm, didx_v, ones_v, deg_sh):
    cid = lax.axis_index("c")
    tid = lax.axis_index("s")
    pltpu.sync_copy(dst_hbm.at[pl.ds((cid * NS + tid) * NBH, NBH)], didx_v)
    pltpu.sync_copy(ones_hbm, ones_v)
    pltpu.sync_copy(zeros_hbm, deg_sh.at[pl.ds(tid * NROWS_T, NROWS_T)])
    plsc.subcore_barrier()

    def body(b, _):
        pltpu.sync_copy(ones_v, deg_sh.at[didx_v.at[b]], add=True)
        return 0

    lax.fori_loop(0, NBH, body, 0)
    plsc.subcore_barrier()
    pltpu.sync_copy(deg_sh.at[pl.ds(tid * NROWS_T, NROWS_T)],
                    out_hbm.at[pl.ds(cid * NPAD + tid * NROWS_T, NROWS_T)])


def _prep_body(parts_ref, x_ref, xs_ref, dinv_ref):
    p = parts_ref[:, :, 0:1]
    deg = p[0] + p[1] + 1.0
    dinv = lax.rsqrt(deg)
    xsb = x_ref[...] * dinv
    for c in range(C):
        xs_ref[c] = xsb[:, c * F:(c + 1) * F]
    dinv_ref[...] = dinv


def _prep(parts, x):
    return pl.pallas_call(
        _prep_body,
        grid=(NPAD // ROW_BLK,),
        in_specs=[
            pl.BlockSpec((NC, ROW_BLK, F), lambda i: (0, i, 0)),
            pl.BlockSpec((ROW_BLK, D), lambda i: (i, 0)),
        ],
        out_specs=[
            pl.BlockSpec((C, ROW_BLK, F), lambda i: (0, i, 0)),
            pl.BlockSpec((ROW_BLK, 1), lambda i: (i, 0)),
        ],
        out_shape=[
            jax.ShapeDtypeStruct((C, NPAD, F), jnp.float32),
            jax.ShapeDtypeStruct((NPAD, 1), jnp.float32),
        ],
    )(parts, x)


EPS = E_PAD // NS
EBS = 64
NB = EPS // EBS
NB8 = 16
NBUF = 3


@functools.partial(
    pl.kernel,
    out_type=jax.ShapeDtypeStruct((C * NPAD, F), jnp.float32),
    mesh=_mesh,
    scratch_types=[
        pltpu.VMEM((NB8, EBS), jnp.int32),
        pltpu.VMEM((NB8, EBS), jnp.int32),
        pltpu.VMEM((NBUF, EBS, F), jnp.float32),
        pltpu.VMEM_SHARED((NPAD, F), jnp.float32),
        pltpu.SemaphoreType.DMA,
        pltpu.SemaphoreType.DMA,
        pltpu.SemaphoreType.DMA,
    ],
)
def _scatter_k(xs_hbm, srcadj_hbm, dst_hbm, zeros_hbm, out_hbm,
               src_v, dst_v, rows_v, z_sh, g0, g1, g2):
    cid = lax.axis_index("c")
    tid = lax.axis_index("s")
    gs = (g0, g1, g2)
    CPS = C // NC

    for cc in range(CPS):
        c = cid * CPS + cc
        pltpu.sync_copy(zeros_hbm, z_sh.at[pl.ds(tid * NROWS_T, NROWS_T)])
        plsc.subcore_barrier()

        def block(h, _):
            pltpu.sync_copy(
                srcadj_hbm.at[pl.ds(c * (E_PAD // EBS) + tid * NB + h * NB8,
                                    NB8)],
                src_v)
            pltpu.sync_copy(dst_hbm.at[pl.ds(tid * NB + h * NB8, NB8)], dst_v)

            def wait_gather(b, buf):
                pltpu.make_async_copy(xs_hbm.at[src_v.at[b]],
                                      rows_v.at[buf], gs[buf]).wait()

            for b in range(NBUF):
                pltpu.async_copy(xs_hbm.at[src_v.at[b]],
                                 rows_v.at[b], gs[b])
            for b in range(NB8):
                buf = b % NBUF
                wait_gather(b, buf)
                pltpu.sync_copy(rows_v.at[buf], z_sh.at[dst_v.at[b]],
                                add=True)
                if b + NBUF < NB8:
                    pltpu.async_copy(xs_hbm.at[src_v.at[b + NBUF]],
                                     rows_v.at[buf], gs[buf])
            return 0

        lax.fori_loop(0, NB // NB8, block, 0)
        plsc.subcore_barrier()
        pltpu.sync_copy(
            z_sh.at[pl.ds(tid * NROWS_T, NROWS_T)],
            out_hbm.at[pl.ds(c * NPAD + tid * NROWS_T, NROWS_T)])


def _head_body(z_ref, xs_ref, dinv_ref, x_ref,
               wcat_ref, bcat_ref, wlt_ref, wlb_ref, bl_ref, out_ref):
    y = jnp.concatenate(
        [z_ref[c] + xs_ref[c] for c in range(C)], axis=-1) * dinv_ref[...]
    g = jnp.maximum(jnp.dot(y, wcat_ref[...], preferred_element_type=jnp.float32)
                    + bcat_ref[...], 0.0)
    h = (jnp.dot(g, wlt_ref[...], preferred_element_type=jnp.float32)
         + jnp.dot(x_ref[...], wlb_ref[...], preferred_element_type=jnp.float32)
         + bl_ref[...])
    out_ref[...] = jnp.maximum(h, 0.0).astype(jnp.bfloat16)


def _head(z, xs, dinv, x, wcat, bcat, wl_top, wl_bot, bl):
    return pl.pallas_call(
        _head_body,
        grid=(NPAD // ROW_BLK,),
        in_specs=[
            pl.BlockSpec((C, ROW_BLK, F), lambda i: (0, i, 0)),
            pl.BlockSpec((C, ROW_BLK, F), lambda i: (0, i, 0)),
            pl.BlockSpec((ROW_BLK, 1), lambda i: (i, 0)),
            pl.BlockSpec((ROW_BLK, D), lambda i: (i, 0)),
            pl.BlockSpec((D, D), lambda i: (0, 0)),
            pl.BlockSpec((1, D), lambda i: (0, 0)),
            pl.BlockSpec((D, H2), lambda i: (0, 0)),
            pl.BlockSpec((D, H2), lambda i: (0, 0)),
            pl.BlockSpec((1, H2), lambda i: (0, 0)),
        ],
        out_specs=pl.BlockSpec((ROW_BLK, H2), lambda i: (i, 0)),
        out_shape=jax.ShapeDtypeStruct((NPAD, H2), jnp.bfloat16),
    )(z, xs, dinv, x, wcat, bcat.reshape(1, D), wl_top, wl_bot,
      bl.reshape(1, H2))


EB_T = E2_PAD // NW
H2I = H2 // 2
NBUF2 = 4
LOOK = 2


@functools.partial(
    pl.kernel,
    out_type=[
        jax.ShapeDtypeStruct((E2_PAD, H2I), jnp.int32),
        jax.ShapeDtypeStruct((E2_PAD, H2I), jnp.int32),
    ],
    mesh=_mesh,
    scratch_types=[
        pltpu.VMEM((NB2, EB), jnp.int32),
        pltpu.VMEM((NB2, EB), jnp.int32),
        pltpu.VMEM((NBUF2, EB, H2I), jnp.int32),
        pltpu.VMEM((NBUF2, EB, H2I), jnp.int32),
    ] + [pltpu.SemaphoreType.DMA] * 16,
)
def _score_k(h2_hbm, sidx_hbm, didx_hbm, hs_hbm, hd_hbm,
             sidx_v, didx_v, srows_v, drows_v, *sems):
    wid = lax.axis_index("s") * NC + lax.axis_index("c")
    pltpu.sync_copy(sidx_hbm.at[pl.ds(wid * NB2, NB2)], sidx_v)
    pltpu.sync_copy(didx_hbm.at[pl.ds(wid * NB2, NB2)], didx_v)
    gssem = sems[0:4]
    gdsem = sems[4:8]
    wssem = sems[8:12]
    wdsem = sems[12:16]
    base = wid * EB_T

    def g_src(b, s):
        return pltpu.make_async_copy(h2_hbm.at[sidx_v.at[b]],
                                     srows_v.at[s], gssem[s])

    def g_dst(b, s):
        return pltpu.make_async_copy(h2_hbm.at[didx_v.at[b]],
                                     drows_v.at[s], gdsem[s])

    def w_src(b, s):
        return pltpu.make_async_copy(srows_v.at[s],
                                     hs_hbm.at[pl.ds(base + b * EB, EB)],
                                     wssem[s])

    def w_dst(b, s):
        return pltpu.make_async_copy(drows_v.at[s],
                                     hd_hbm.at[pl.ds(base + b * EB, EB)],
                                     wdsem[s])

    for b in range(LOOK):
        g_src(b, b).start()
        g_dst(b, b).start()

    def body(j, _):
        for k in range(NBUF2):
            b = NBUF2 * j + k
            g_src(b, k).wait()
            w_src(b, k).start()
            g_dst(b, k).wait()
            w_dst(b, k).start()
            sq = (k + LOOK) % NBUF2

            @pl.when(b >= LOOK)
            def _():
                w_src(b - LOOK, sq).wait()
                w_dst(b - LOOK, sq).wait()

            @pl.when(b + LOOK < NB2)
            def _():
                g_src(b + LOOK, sq).start()
                g_dst(b + LOOK, sq).start()
        return 0

    lax.fori_loop(0, NB2 // NBUF2, body, 0)
    for b in range(NB2 - LOOK, NB2):
        w_src(b, b % NBUF2).wait()
        w_dst(b, b % NBUF2).wait()


EROW = 2048


def _edot_body(hs_ref, hd_ref, out_ref):
    ones = jnp.ones((H2I, 1), jnp.float32)
    acc = jnp.zeros((EROW, 1), jnp.float32)
    s = hs_ref[...]
    d = hd_ref[...]
    for part in (jnp.int32(-65536), None):
        if part is None:
            sf = lax.bitcast_convert_type(lax.shift_left(s, 16), jnp.float32)
            df = lax.bitcast_convert_type(lax.shift_left(d, 16), jnp.float32)
        else:
            sf = lax.bitcast_convert_type(s & part, jnp.float32)
            df = lax.bitcast_convert_type(d & part, jnp.float32)
        acc = acc + jnp.dot(sf * df, ones, preferred_element_type=jnp.float32)
    out_ref[...] = acc


def _edot(hs, hd):
    return pl.pallas_call(
        _edot_body,
        grid=(E2_PAD // EROW,),
        in_specs=[
            pl.BlockSpec((EROW, H2I), lambda i: (i, 0)),
            pl.BlockSpec((EROW, H2I), lambda i: (i, 0)),
        ],
        out_specs=pl.BlockSpec((EROW, 1), lambda i: (i, 0)),
        out_shape=jax.ShapeDtypeStruct((E2_PAD, 1), jnp.float32),
    )(hs, hd)


def kernel(pos_edge_index, neg_edge_index, x, train_pos_edge_index,
           W1, b1, W2, b2, W3, b3, W4, b4, Wl, bl):
    i32 = jnp.int32
    src = train_pos_edge_index[0].astype(i32)
    dst = train_pos_edge_index[1].astype(i32)
    padN = jnp.full((E_PAD - E_TRAIN,), N, i32)
    src_p = jnp.concatenate([src, padN])
    dst_p = jnp.concatenate([dst, padN])
    srcadj = (src_p[None, :] + (jnp.arange(C, dtype=i32) * NPAD)[:, None])
    srcadj = srcadj.reshape(C * (E_PAD // EBS), EBS)
    dst2d = dst_p.reshape(E_PAD // 128, 128)
    dst2d_s = dst_p.reshape(E_PAD // EBS, EBS)
    zeros_rows = jnp.zeros((NROWS_T, F), jnp.float32)

    ones_rows = jnp.ones((128, F), jnp.float32)
    parts = _hist_k(dst2d, ones_rows, zeros_rows).reshape(NC, NPAD, F)
    x_pad = jnp.pad(x, ((0, NPAD - N), (0, 0)))
    xs, dinv = _prep(parts, x_pad)

    z_flat = _scatter_k(xs.reshape(C * NPAD, F), srcadj, dst2d_s, zeros_rows)
    z = z_flat.reshape(C, NPAD, F)

    wcat = jnp.concatenate([W1, W2, W3, W4], axis=1)
    bcat = jnp.concatenate([b1, b2, b3, b4])
    h2 = _head(z, xs, dinv, x_pad, wcat, bcat, Wl[:D], Wl[D:], bl)

    h2i = lax.bitcast_convert_type(h2.reshape(NPAD, H2 // 2, 2), i32)

    te = jnp.concatenate([pos_edge_index, neg_edge_index], axis=-1).astype(i32)
    pad0 = jnp.zeros((E2_PAD - E_SCORE,), i32)
    te_src = jnp.concatenate([te[0], pad0]).reshape(E2_PAD // EB, EB)
    te_dst = jnp.concatenate([te[1], pad0]).reshape(E2_PAD // EB, EB)
    hs, hd = _score_k(h2i, te_src, te_dst)
    scores = _edot(hs, hd)
    return scores[:E_SCORE, 0]


# speedup vs baseline: 1.0248x; 1.0049x over previous
import functools

import jax
import jax.numpy as jnp
from jax import lax
from jax.experimental import pallas as pl
from jax.experimental.pallas import tpu as pltpu
from jax.experimental.pallas import tpu_sc as plsc

N = 10000
NPAD = 10240
D = 1024
H2 = 512
F = 128
C = D // F
NC, NS, L = 2, 16, 16
NW = NC * NS
E_TRAIN = 160000
E_PAD = 163840
E_SCORE = 100000
EB = 32
NB2 = 104
E2_PAD = 32 * EB * NB2

ROW_BLK = 640

_mesh = plsc.VectorSubcoreMesh(core_axis_name="c", subcore_axis_name="s")


NBH = E_PAD // (NC * NS * 128)
NROWS_T = NPAD // NS


@functools.partial(
    pl.kernel,
    out_type=jax.ShapeDtypeStruct((NC * NPAD, F), jnp.float32),
    mesh=_mesh,
    scratch_types=[
        pltpu.VMEM((NBH, 128), jnp.int32),
        pltpu.VMEM((128, F), jnp.float32),
        pltpu.VMEM_SHARED((NPAD, F), jnp.float32),
    ],
)
def _hist_k(dst_hbm, ones_hbm, zeros_hbm, out_hbm, didx_v, ones_v, deg_sh):
    cid = lax.axis_index("c")
    tid = lax.axis_index("s")
    pltpu.sync_copy(dst_hbm.at[pl.ds((cid * NS + tid) * NBH, NBH)], didx_v)
    pltpu.sync_copy(ones_hbm, ones_v)
    pltpu.sync_copy(zeros_hbm, deg_sh.at[pl.ds(tid * NROWS_T, NROWS_T)])
    plsc.subcore_barrier()

    def body(b, _):
        pltpu.sync_copy(ones_v, deg_sh.at[didx_v.at[b]], add=True)
        return 0

    lax.fori_loop(0, NBH, body, 0)
    plsc.subcore_barrier()
    pltpu.sync_copy(deg_sh.at[pl.ds(tid * NROWS_T, NROWS_T)],
                    out_hbm.at[pl.ds(cid * NPAD + tid * NROWS_T, NROWS_T)])


def _prep_body(parts_ref, x_ref, xs_ref, dinv_ref):
    p = parts_ref[:, :, 0:1]
    deg = p[0] + p[1] + 1.0
    dinv = lax.rsqrt(deg)
    xsb = x_ref[...] * dinv
    for c in range(C):
        xs_ref[c] = xsb[:, c * F:(c + 1) * F]
    dinv_ref[...] = dinv


def _prep(parts, x):
    return pl.pallas_call(
        _prep_body,
        grid=(NPAD // ROW_BLK,),
        in_specs=[
            pl.BlockSpec((NC, ROW_BLK, F), lambda i: (0, i, 0)),
            pl.BlockSpec((ROW_BLK, D), lambda i: (i, 0)),
        ],
        out_specs=[
            pl.BlockSpec((C, ROW_BLK, F), lambda i: (0, i, 0)),
            pl.BlockSpec((ROW_BLK, 1), lambda i: (i, 0)),
        ],
        out_shape=[
            jax.ShapeDtypeStruct((C, NPAD, F), jnp.float32),
            jax.ShapeDtypeStruct((NPAD, 1), jnp.float32),
        ],
    )(parts, x)


EPS = E_PAD // NS
EBS = 64
NB = EPS // EBS
NB8 = 80
NBUF = 2


@functools.partial(
    pl.kernel,
    out_type=jax.ShapeDtypeStruct((C * NPAD, F), jnp.float32),
    mesh=_mesh,
    scratch_types=[
        pltpu.VMEM((NB8, EBS), jnp.int32),
        pltpu.VMEM((NB8, EBS), jnp.int32),
        pltpu.VMEM((NBUF, EBS, F), jnp.float32),
        pltpu.VMEM_SHARED((NPAD, F), jnp.float32),
        pltpu.SemaphoreType.DMA,
        pltpu.SemaphoreType.DMA,
        pltpu.SemaphoreType.DMA,
    ],
)
def _scatter_k(xs_hbm, srcadj_hbm, dst_hbm, zeros_hbm, out_hbm,
               src_v, dst_v, rows_v, z_sh, g0, g1, g2):
    cid = lax.axis_index("c")
    tid = lax.axis_index("s")
    gs = (g0, g1, g2)
    CPS = C // NC

    for cc in range(CPS):
        c = cid * CPS + cc
        pltpu.sync_copy(zeros_hbm, z_sh.at[pl.ds(tid * NROWS_T, NROWS_T)])
        plsc.subcore_barrier()

        def block(h, _):
            pltpu.sync_copy(
                srcadj_hbm.at[pl.ds(c * (E_PAD // EBS) + tid * NB + h * NB8,
                                    NB8)],
                src_v)
            pltpu.sync_copy(dst_hbm.at[pl.ds(tid * NB + h * NB8, NB8)], dst_v)

            def wait_gather(b, buf):
                pltpu.make_async_copy(xs_hbm.at[src_v.at[b]],
                                      rows_v.at[buf], gs[buf]).wait()

            for b in range(NBUF):
                pltpu.async_copy(xs_hbm.at[src_v.at[b]],
                                 rows_v.at[b], gs[b])
            for b in range(NB8):
                buf = b % NBUF
                wait_gather(b, buf)
                pltpu.sync_copy(rows_v.at[buf], z_sh.at[dst_v.at[b]],
                                add=True)
                if b + NBUF < NB8:
                    pltpu.async_copy(xs_hbm.at[src_v.at[b + NBUF]],
                                     rows_v.at[buf], gs[buf])
            return 0

        lax.fori_loop(0, NB // NB8, block, 0)
        plsc.subcore_barrier()
        pltpu.sync_copy(
            z_sh.at[pl.ds(tid * NROWS_T, NROWS_T)],
            out_hbm.at[pl.ds(c * NPAD + tid * NROWS_T, NROWS_T)])


def _head_body(z_ref, xs_ref, dinv_ref, x_ref,
               wcat_ref, bcat_ref, wlt_ref, wlb_ref, bl_ref, out_ref):
    y = jnp.concatenate(
        [z_ref[c] + xs_ref[c] for c in range(C)], axis=-1) * dinv_ref[...]
    g = jnp.maximum(jnp.dot(y, wcat_ref[...], preferred_element_type=jnp.float32)
                    + bcat_ref[...], 0.0)
    h = (jnp.dot(g, wlt_ref[...], preferred_element_type=jnp.float32)
         + jnp.dot(x_ref[...], wlb_ref[...], preferred_element_type=jnp.float32)
         + bl_ref[...])
    out_ref[...] = jnp.maximum(h, 0.0).astype(jnp.bfloat16)


def _head(z, xs, dinv, x, wcat, bcat, wl_top, wl_bot, bl):
    return pl.pallas_call(
        _head_body,
        grid=(NPAD // ROW_BLK,),
        in_specs=[
            pl.BlockSpec((C, ROW_BLK, F), lambda i: (0, i, 0)),
            pl.BlockSpec((C, ROW_BLK, F), lambda i: (0, i, 0)),
            pl.BlockSpec((ROW_BLK, 1), lambda i: (i, 0)),
            pl.BlockSpec((ROW_BLK, D), lambda i: (i, 0)),
            pl.BlockSpec((D, D), lambda i: (0, 0)),
            pl.BlockSpec((1, D), lambda i: (0, 0)),
            pl.BlockSpec((D, H2), lambda i: (0, 0)),
            pl.BlockSpec((D, H2), lambda i: (0, 0)),
            pl.BlockSpec((1, H2), lambda i: (0, 0)),
        ],
        out_specs=pl.BlockSpec((ROW_BLK, H2), lambda i: (i, 0)),
        out_shape=jax.ShapeDtypeStruct((NPAD, H2), jnp.bfloat16),
    )(z, xs, dinv, x, wcat, bcat.reshape(1, D), wl_top, wl_bot,
      bl.reshape(1, H2))


EB_T = E2_PAD // NW
H2I = H2 // 2
NBUF2 = 4
LOOK = 2


@functools.partial(
    pl.kernel,
    out_type=[
        jax.ShapeDtypeStruct((E2_PAD, H2I), jnp.int32),
        jax.ShapeDtypeStruct((E2_PAD, H2I), jnp.int32),
    ],
    mesh=_mesh,
    scratch_types=[
        pltpu.VMEM((NB2, EB), jnp.int32),
        pltpu.VMEM((NB2, EB), jnp.int32),
        pltpu.VMEM((NBUF2, EB, H2I), jnp.int32),
        pltpu.VMEM((NBUF2, EB, H2I), jnp.int32),
    ] + [pltpu.SemaphoreType.DMA] * 16,
)
def _score_k(h2_hbm, sidx_hbm, didx_hbm, hs_hbm, hd_hbm,
             sidx_v, didx_v, srows_v, drows_v, *sems):
    wid = lax.axis_index("s") * NC + lax.axis_index("c")
    pltpu.sync_copy(sidx_hbm.at[pl.ds(wid * NB2, NB2)], sidx_v)
    pltpu.sync_copy(didx_hbm.at[pl.ds(wid * NB2, NB2)], didx_v)
    gssem = sems[0:4]
    gdsem = sems[4:8]
    wssem = sems[8:12]
    wdsem = sems[12:16]
    base = wid * EB_T

    def g_src(b, s):
        return pltpu.make_async_copy(h2_hbm.at[sidx_v.at[b]],
                                     srows_v.at[s], gssem[s])

    def g_dst(b, s):
        return pltpu.make_async_copy(h2_hbm.at[didx_v.at[b]],
                                     drows_v.at[s], gdsem[s])

    def w_src(b, s):
        return pltpu.make_async_copy(srows_v.at[s],
                                     hs_hbm.at[pl.ds(base + b * EB, EB)],
                                     wssem[s])

    def w_dst(b, s):
        return pltpu.make_async_copy(drows_v.at[s],
                                     hd_hbm.at[pl.ds(base + b * EB, EB)],
                                     wdsem[s])

    for b in range(LOOK):
        g_src(b, b).start()
        g_dst(b, b).start()

    def body(j, _):
        for k in range(NBUF2):
            b = NBUF2 * j + k
            g_src(b, k).wait()
            w_src(b, k).start()
            g_dst(b, k).wait()
            w_dst(b, k).start()
            sq = (k + LOOK) % NBUF2

            @pl.when(b >= LOOK)
            def _():
                w_src(b - LOOK, sq).wait()
                w_dst(b - LOOK, sq).wait()

            @pl.when(b + LOOK < NB2)
            def _():
                g_src(b + LOOK, sq).start()
                g_dst(b + LOOK, sq).start()
        return 0

    lax.fori_loop(0, NB2 // NBUF2, body, 0)
    for b in range(NB2 - LOOK, NB2):
        w_src(b, b % NBUF2).wait()
        w_dst(b, b % NBUF2).wait()


EROW = 2048


def _edot_body(hs_ref, hd_ref, out_ref):
    ones = jnp.ones((H2I, 1), jnp.float32)
    acc = jnp.zeros((EROW, 1), jnp.float32)
    s = hs_ref[...]
    d = hd_ref[...]
    for part in (jnp.int32(-65536), None):
        if part is None:
            sf = lax.bitcast_convert_type(lax.shift_left(s, 16), jnp.float32)
            df = lax.bitcast_convert_type(lax.shift_left(d, 16), jnp.float32)
        else:
            sf = lax.bitcast_convert_type(s & part, jnp.float32)
            df = lax.bitcast_convert_type(d & part, jnp.float32)
        acc = acc + jnp.dot(sf * df, ones, preferred_element_type=jnp.float32)
    out_ref[...] = acc


def _edot(hs, hd):
    return pl.pallas_call(
        _edot_body,
        grid=(E2_PAD // EROW,),
        in_specs=[
            pl.BlockSpec((EROW, H2I), lambda i: (i, 0)),
            pl.BlockSpec((EROW, H2I), lambda i: (i, 0)),
        ],
        out_specs=pl.BlockSpec((EROW, 1), lambda i: (i, 0)),
        out_shape=jax.ShapeDtypeStruct((E2_PAD, 1), jnp.float32),
    )(hs, hd)


def kernel(pos_edge_index, neg_edge_index, x, train_pos_edge_index,
           W1, b1, W2, b2, W3, b3, W4, b4, Wl, bl):
    i32 = jnp.int32
    src = train_pos_edge_index[0].astype(i32)
    dst = train_pos_edge_index[1].astype(i32)
    padN = jnp.full((E_PAD - E_TRAIN,), N, i32)
    src_p = jnp.concatenate([src, padN])
    dst_p = jnp.concatenate([dst, padN])
    srcadj = (src_p[None, :] + (jnp.arange(C, dtype=i32) * NPAD)[:, None])
    srcadj = srcadj.reshape(C * (E_PAD // EBS), EBS)
    dst2d = dst_p.reshape(E_PAD // 128, 128)
    dst2d_s = dst_p.reshape(E_PAD // EBS, EBS)
    zeros_rows = jnp.zeros((NROWS_T, F), jnp.float32)

    ones_rows = jnp.ones((128, F), jnp.float32)
    parts = _hist_k(dst2d, ones_rows, zeros_rows).reshape(NC, NPAD, F)
    x_pad = jnp.pad(x, ((0, NPAD - N), (0, 0)))
    xs, dinv = _prep(parts, x_pad)

    z_flat = _scatter_k(xs.reshape(C * NPAD, F), srcadj, dst2d_s, zeros_rows)
    z = z_flat.reshape(C, NPAD, F)

    wcat = jnp.concatenate([W1, W2, W3, W4], axis=1)
    bcat = jnp.concatenate([b1, b2, b3, b4])
    h2 = _head(z, xs, dinv, x_pad, wcat, bcat, Wl[:D], Wl[D:], bl)

    h2i = lax.bitcast_convert_type(h2.reshape(NPAD, H2 // 2, 2), i32)

    te = jnp.concatenate([pos_edge_index, neg_edge_index], axis=-1).astype(i32)
    pad0 = jnp.zeros((E2_PAD - E_SCORE,), i32)
    te_src = jnp.concatenate([te[0], pad0]).reshape(E2_PAD // EB, EB)
    te_dst = jnp.concatenate([te[1], pad0]).reshape(E2_PAD // EB, EB)
    hs, hd = _score_k(h2i, te_src, te_dst)
    scores = _edot(hs, hd)
    return scores[:E_SCORE, 0]
